# Initial kernel scaffold; baseline (speedup 1.0000x reference)
import jax
import jax.numpy as jnp
from jax.experimental import pallas as pl


def kernel(price, ctr, item_id, cate_id, W_num, b_num, table_items, table_cates):
    raise NotImplementedError("write your pallas kernel here")


# speedup vs baseline: 1.9506x; 1.9506x over previous
import functools

import jax
import jax.numpy as jnp
from jax import lax
from jax.experimental import pallas as pl
from jax.experimental.pallas import tpu as pltpu
from jax.experimental.pallas import tpu_sc as plsc

B, L = 16384, 50
N = B * L
NUM_DIM = 8
ITEM_DIM = 50
CATE_DIM = 12
OUT_DIM = 2 * NUM_DIM + ITEM_DIM + CATE_DIM
CATE_ROWS = 1001

NW = 32
PER_W = N // NW
C = 512
NCHUNK = PER_W // C
GID = C // 128
NG = C // 16

_mesh = plsc.VectorSubcoreMesh(core_axis_name="c", subcore_axis_name="s")


@functools.partial(
    pl.kernel,
    out_type=jax.ShapeDtypeStruct((N * OUT_DIM,), jnp.float32),
    mesh=_mesh,
    compiler_params=pltpu.CompilerParams(
        needs_layout_passes=False, use_tc_tiling_on_sc=False),
    scratch_types=[
        pltpu.VMEM((C,), jnp.float32),
        pltpu.VMEM((C,), jnp.float32),
        pltpu.VMEM((C,), jnp.int32),
        pltpu.VMEM((GID, 128), jnp.int32),
        pltpu.VMEM((CATE_ROWS * CATE_DIM,), jnp.float32),
        pltpu.VMEM((16, 16), jnp.float32),
        pltpu.VMEM((C, ITEM_DIM), jnp.float32),
        pltpu.VMEM((C * OUT_DIM,), jnp.float32),
        pltpu.SemaphoreType.DMA,
    ],
)
def _encoder(price_h, ctr_h, cate_h, item2_h, wb_h, tcat_h, titem_h, out_h,
             price_v, ctr_v, cate_v, idx_v, ctab_v, wb_v, item_b, out_s, sem):
    wid = lax.axis_index("s") * 2 + lax.axis_index("c")
    base = wid * PER_W

    pltpu.sync_copy(tcat_h, ctab_v)
    pltpu.sync_copy(wb_h, wb_v)

    def chunk_body(ch, carry):
        row0 = base + ch * C
        pltpu.sync_copy(price_h.at[pl.ds(row0, C)], price_v)
        pltpu.sync_copy(ctr_h.at[pl.ds(row0, C)], ctr_v)
        pltpu.sync_copy(cate_h.at[pl.ds(row0, C)], cate_v)
        pltpu.sync_copy(item2_h.at[pl.ds(wid * (PER_W // 128) + ch * GID, GID)],
                        idx_v)

        copies = [
            pltpu.async_copy(
                titem_h.at[idx_v.at[j]],
                item_b.at[pl.ds(j * 128, 128)],
                sem,
            )
            for j in range(GID)
        ]

        def group_body(g, carry2):
            r0 = g * 16
            fbase = (lax.iota(jnp.int32, 16) + r0) * OUT_DIM
            p = price_v[pl.ds(r0, 16)]
            c = ctr_v[pl.ds(r0, 16)]
            pm = p == p
            cm = c == c
            pc = jnp.where(pm, p, 0.0)
            cc = jnp.where(cm, c, 0.0)
            for d in range(NUM_DIM):
                w = wb_v[2 * d, :]
                bb = wb_v[2 * d + 1, :]
                vp = jnp.maximum(pc * w + bb, 0.0)
                vp = jnp.where(pm, vp, 0.0)
                plsc.store_scatter(out_s, [fbase + d], vp)
                vc = jnp.maximum(cc * w + bb, 0.0)
                vc = jnp.where(cm, vc, 0.0)
                plsc.store_scatter(out_s, [fbase + (NUM_DIM + d)], vc)
            cid = cate_v[pl.ds(r0, 16)] * CATE_DIM
            for k in range(CATE_DIM):
                col = plsc.load_gather(ctab_v, [cid + k])
                plsc.store_scatter(
                    out_s, [fbase + (2 * NUM_DIM + ITEM_DIM + k)], col)
            return carry2

        lax.fori_loop(0, NG, group_body, 0)

        for cp in copies:
            cp.wait()

        def row_body(i, carry3):
            dst = i * OUT_DIM + 2 * NUM_DIM
            for off in (0, 16, 32, 34):
                out_s[pl.ds(dst + off, 16)] = item_b[i, pl.ds(off, 16)]
            return carry3

        lax.fori_loop(0, C, row_body, 0)

        pltpu.sync_copy(out_s, out_h.at[pl.ds(row0 * OUT_DIM, C * OUT_DIM)])
        return carry

    lax.fori_loop(0, NCHUNK, chunk_body, 0)


def kernel(price, ctr, item_id, cate_id, W_num, b_num, table_items,
           table_cates):
    wb = jnp.stack([W_num[0], b_num], axis=1).reshape(16)
    wb = jnp.broadcast_to(wb[:, None], (16, 16))
    out = _encoder(
        price.reshape(N),
        ctr.reshape(N),
        cate_id.reshape(N),
        item_id.reshape(N // 128, 128),
        wb,
        table_cates.reshape(CATE_ROWS * CATE_DIM),
        table_items,
    )
    return out.reshape(B, L, OUT_DIM)


# speedup vs baseline: 2.1413x; 1.0978x over previous
import functools

import jax
import jax.numpy as jnp
from jax import lax
from jax.experimental import pallas as pl
from jax.experimental.pallas import tpu as pltpu
from jax.experimental.pallas import tpu_sc as plsc

B, L = 16384, 50
NUM_DIM = 8
ITEM_DIM = 50
CATE_DIM = 12
OUT_DIM = 2 * NUM_DIM + ITEM_DIM + CATE_DIM
CATE_ROWS = 1001
ITEM_ROWS = 1000001
RT_ROWS = 1000064
FULL_COLS = ITEM_ROWS // 128
TAIL0 = FULL_COLS * 128

NW = 32
COL_UNITS = 245
NBLK = B // 128
UNITS = L * NBLK // NW

_mesh = plsc.VectorSubcoreMesh(core_axis_name="c", subcore_axis_name="s")
_params = pltpu.CompilerParams(needs_layout_passes=False)


@functools.partial(
    pl.kernel,
    out_type=jax.ShapeDtypeStruct((RT_ROWS, 128), jnp.float32),
    mesh=_mesh,
    compiler_params=_params,
    scratch_types=[
        pltpu.VMEM((ITEM_DIM, 128), jnp.float32),
        pltpu.VMEM((128, 128), jnp.float32),
        pltpu.VMEM((128, ITEM_DIM), jnp.float32),
    ],
)
def _relayout(tT_h, tail_h, rt_h, buf, tb, tlb):
    wid = lax.axis_index("s") * 2 + lax.axis_index("c")

    def col_body(u, carry):
        j = u * NW + wid

        @pl.when(j < FULL_COLS)
        def _():
            pltpu.sync_copy(tT_h.at[:, pl.ds(j * 128, 128)], buf)

            def grp(g, c2):
                iv = lax.iota(jnp.int32, 16) + g * 16
                for c in range(ITEM_DIM):
                    v = buf[c, pl.ds(g * 16, 16)]
                    plsc.store_scatter(
                        tb, [iv, jnp.full((16,), c, jnp.int32)], v)
                return c2

            lax.fori_loop(0, 8, grp, 0)
            pltpu.sync_copy(tb, rt_h.at[pl.ds(j * 128, 128)])

        return carry

    lax.fori_loop(0, COL_UNITS, col_body, 0)

    @pl.when(wid == 0)
    def _():
        pltpu.sync_copy(tail_h, tlb)

        def grp(g, c2):
            iv = lax.iota(jnp.int32, 16) + g * 16
            for c in range(ITEM_DIM):
                cc = jnp.full((16,), c, jnp.int32)
                v = plsc.load_gather(tlb, [iv, cc])
                plsc.store_scatter(tb, [iv, cc], v)
            return c2

        lax.fori_loop(0, 8, grp, 0)
        pltpu.sync_copy(tb, rt_h.at[pl.ds(TAIL0, 128)])


@functools.partial(
    pl.kernel,
    out_type=jax.ShapeDtypeStruct((L, OUT_DIM, B), jnp.float32),
    mesh=_mesh,
    compiler_params=_params,
    scratch_types=[
        pltpu.VMEM((128,), jnp.float32),
        pltpu.VMEM((128,), jnp.float32),
        pltpu.VMEM((128,), jnp.int32),
        pltpu.VMEM((128,), jnp.int32),
        pltpu.VMEM((CATE_DIM, CATE_ROWS), jnp.float32),
        pltpu.VMEM((16, 16), jnp.float32),
        pltpu.VMEM((128, 128), jnp.float32),
        pltpu.VMEM((OUT_DIM, 128), jnp.float32),
        pltpu.SemaphoreType.DMA,
    ],
)
def _encoder(pT_h, cT_h, catT_h, itT_h, ctT_h, wb_h, rt_h, out_h,
             pv, cv, gv, iv_, ctab, wbv, irows, slab, sem):
    wid = lax.axis_index("s") * 2 + lax.axis_index("c")

    pltpu.sync_copy(ctT_h, ctab)
    pltpu.sync_copy(wb_h, wbv)

    def unit_body(u, carry):
        unit = u * NW + wid
        l = unit // NBLK
        b0 = (unit % NBLK) * 128

        pltpu.sync_copy(itT_h.at[l, pl.ds(b0, 128)], iv_)
        gather = pltpu.async_copy(rt_h.at[iv_], irows, sem)
        pltpu.sync_copy(pT_h.at[l, pl.ds(b0, 128)], pv)
        pltpu.sync_copy(cT_h.at[l, pl.ds(b0, 128)], cv)
        pltpu.sync_copy(catT_h.at[l, pl.ds(b0, 128)], gv)

        def grp_a(g, c2):
            s = g * 16
            p = pv[pl.ds(s, 16)]
            c = cv[pl.ds(s, 16)]
            pm = p == p
            cm = c == c
            pc = jnp.where(pm, p, 0.0)
            cc = jnp.where(cm, c, 0.0)
            for d in range(NUM_DIM):
                w = wbv[2 * d, :]
                bb = wbv[2 * d + 1, :]
                vp = jnp.maximum(pc * w + bb, 0.0)
                slab[d, pl.ds(s, 16)] = jnp.where(pm, vp, 0.0)
                vc = jnp.maximum(cc * w + bb, 0.0)
                slab[NUM_DIM + d, pl.ds(s, 16)] = jnp.where(cm, vc, 0.0)
            ids = gv[pl.ds(s, 16)]
            for k in range(CATE_DIM):
                col = plsc.load_gather(
                    ctab, [jnp.full((16,), k, jnp.int32), ids])
                slab[2 * NUM_DIM + ITEM_DIM + k, pl.ds(s, 16)] = col
            return c2

        lax.fori_loop(0, 8, grp_a, 0)
        gather.wait()

        def grp_b(g, c2):
            s = g * 16
            iv16 = lax.iota(jnp.int32, 16) + s
            for c in range(ITEM_DIM):
                v = plsc.load_gather(
                    irows, [iv16, jnp.full((16,), c, jnp.int32)])
                slab[2 * NUM_DIM + c, pl.ds(s, 16)] = v
            return c2

        lax.fori_loop(0, 8, grp_b, 0)
        pltpu.sync_copy(slab, out_h.at[l, :, pl.ds(b0, 128)])
        return carry

    lax.fori_loop(0, UNITS, unit_body, 0)


def kernel(price, ctr, item_id, cate_id, W_num, b_num, table_items,
           table_cates):
    tail = jnp.pad(
        lax.slice(table_items, (TAIL0, 0), (ITEM_ROWS, ITEM_DIM)),
        ((0, 128 - (ITEM_ROWS - TAIL0)), (0, 0)))
    rtab = _relayout(table_items.T, tail)
    wb = jnp.stack([W_num[0], b_num], axis=1).reshape(16)
    wb = jnp.broadcast_to(wb[:, None], (16, 16))
    outT = _encoder(price.T, ctr.T, cate_id.T, item_id.T,
                    table_cates.T, wb, rtab)
    return outT.transpose(2, 0, 1)


# speedup vs baseline: 2.7944x; 1.3050x over previous
import functools

import jax
import jax.numpy as jnp
from jax import lax
from jax.experimental import pallas as pl
from jax.experimental.pallas import tpu as pltpu
from jax.experimental.pallas import tpu_sc as plsc

B, L = 16384, 50
NUM_DIM = 8
ITEM_DIM = 50
CATE_DIM = 12
OUT_DIM = 2 * NUM_DIM + ITEM_DIM + CATE_DIM
CATE_ROWS = 1001
ITEM_ROWS = 1000001
RT_ROWS = 1000064
FULL_COLS = ITEM_ROWS // 128
TAIL0 = FULL_COLS * 128

NW = 32
COL_UNITS = 245
NBLK = B // 128
UNITS = L * NBLK // NW

_mesh = plsc.VectorSubcoreMesh(core_axis_name="c", subcore_axis_name="s")
_params = pltpu.CompilerParams(needs_layout_passes=False)


RL_BLK = 4096


def _relayout_body(t_ref, rt_ref):
    rt_ref[...] = jnp.pad(t_ref[...], ((0, 0), (0, 128 - ITEM_DIM)))


def _relayout(table_items):
    grid = (RT_ROWS + RL_BLK - 1) // RL_BLK
    return pl.pallas_call(
        _relayout_body,
        grid=(grid,),
        in_specs=[pl.BlockSpec((RL_BLK, ITEM_DIM), lambda i: (i, 0))],
        out_specs=pl.BlockSpec((RL_BLK, 128), lambda i: (i, 0)),
        out_shape=jax.ShapeDtypeStruct((RT_ROWS, 128), jnp.float32),
    )(table_items)


@functools.partial(
    pl.kernel,
    out_type=jax.ShapeDtypeStruct((L, OUT_DIM, B), jnp.float32),
    mesh=_mesh,
    compiler_params=_params,
    scratch_types=[
        pltpu.VMEM((128,), jnp.float32),
        pltpu.VMEM((128,), jnp.float32),
        pltpu.VMEM((128,), jnp.int32),
        pltpu.VMEM((128,), jnp.int32),
        pltpu.VMEM((CATE_DIM, CATE_ROWS), jnp.float32),
        pltpu.VMEM((16, 16), jnp.float32),
        pltpu.VMEM((128, 128), jnp.float32),
        pltpu.VMEM((OUT_DIM, 128), jnp.float32),
        pltpu.SemaphoreType.DMA,
    ],
)
def _encoder(pT_h, cT_h, catT_h, itT_h, ctT_h, wb_h, rt_h, out_h,
             pv, cv, gv, iv_, ctab, wbv, irows, slab, sem):
    wid = lax.axis_index("s") * 2 + lax.axis_index("c")

    pltpu.sync_copy(ctT_h, ctab)
    pltpu.sync_copy(wb_h, wbv)

    def unit_body(u, carry):
        unit = u * NW + wid
        l = unit // NBLK
        b0 = (unit % NBLK) * 128

        pltpu.sync_copy(itT_h.at[l, pl.ds(b0, 128)], iv_)
        gather = pltpu.async_copy(rt_h.at[iv_], irows, sem)
        pltpu.sync_copy(pT_h.at[l, pl.ds(b0, 128)], pv)
        pltpu.sync_copy(cT_h.at[l, pl.ds(b0, 128)], cv)
        pltpu.sync_copy(catT_h.at[l, pl.ds(b0, 128)], gv)

        def grp_a(g, c2):
            s = g * 16
            p = pv[pl.ds(s, 16)]
            c = cv[pl.ds(s, 16)]
            pm = p == p
            cm = c == c
            pc = jnp.where(pm, p, 0.0)
            cc = jnp.where(cm, c, 0.0)
            for d in range(NUM_DIM):
                w = wbv[2 * d, :]
                bb = wbv[2 * d + 1, :]
                vp = jnp.maximum(pc * w + bb, 0.0)
                slab[d, pl.ds(s, 16)] = jnp.where(pm, vp, 0.0)
                vc = jnp.maximum(cc * w + bb, 0.0)
                slab[NUM_DIM + d, pl.ds(s, 16)] = jnp.where(cm, vc, 0.0)
            ids = gv[pl.ds(s, 16)]
            for k in range(CATE_DIM):
                col = plsc.load_gather(
                    ctab, [jnp.full((16,), k, jnp.int32), ids])
                slab[2 * NUM_DIM + ITEM_DIM + k, pl.ds(s, 16)] = col
            return c2

        lax.fori_loop(0, 8, grp_a, 0)
        gather.wait()

        def grp_b(g, c2):
            s = g * 16
            iv16 = lax.iota(jnp.int32, 16) + s
            for c in range(ITEM_DIM):
                v = plsc.load_gather(
                    irows, [iv16, jnp.full((16,), c, jnp.int32)])
                slab[2 * NUM_DIM + c, pl.ds(s, 16)] = v
            return c2

        lax.fori_loop(0, 8, grp_b, 0)
        pltpu.sync_copy(slab, out_h.at[l, :, pl.ds(b0, 128)])
        return carry

    lax.fori_loop(0, UNITS, unit_body, 0)


def kernel(price, ctr, item_id, cate_id, W_num, b_num, table_items,
           table_cates):
    rtab = _relayout(table_items)
    wb = jnp.stack([W_num[0], b_num], axis=1).reshape(16)
    wb = jnp.broadcast_to(wb[:, None], (16, 16))
    outT = _encoder(price.T, ctr.T, cate_id.T, item_id.T,
                    table_cates.T, wb, rtab)
    return outT.transpose(2, 0, 1)


# speedup vs baseline: 2.9789x; 1.0660x over previous
import functools

import jax
import jax.numpy as jnp
from jax import lax
from jax.experimental import pallas as pl
from jax.experimental.pallas import tpu as pltpu
from jax.experimental.pallas import tpu_sc as plsc

B, L = 16384, 50
NUM_DIM = 8
ITEM_DIM = 50
CATE_DIM = 12
OUT_DIM = 2 * NUM_DIM + ITEM_DIM + CATE_DIM
CATE_ROWS = 1001
ITEM_ROWS = 1000001
RT_ROWS = 1000064
FULL_COLS = ITEM_ROWS // 128
TAIL0 = FULL_COLS * 128

NW = 32
COL_UNITS = 245
NBLK = B // 128
UNITS = L * NBLK // NW

_mesh = plsc.VectorSubcoreMesh(core_axis_name="c", subcore_axis_name="s")
_params = pltpu.CompilerParams(needs_layout_passes=False)


RL_BLK = 4096


def _relayout_body(t_ref, rt_ref):
    rt_ref[...] = jnp.pad(t_ref[...], ((0, 0), (0, 128 - ITEM_DIM)))


def _relayout(table_items):
    grid = (RT_ROWS + RL_BLK - 1) // RL_BLK
    return pl.pallas_call(
        _relayout_body,
        grid=(grid,),
        in_specs=[pl.BlockSpec((RL_BLK, ITEM_DIM), lambda i: (i, 0))],
        out_specs=pl.BlockSpec((RL_BLK, 128), lambda i: (i, 0)),
        out_shape=jax.ShapeDtypeStruct((RT_ROWS, 128), jnp.float32),
    )(table_items)


NC_DIM = 2 * NUM_DIM + CATE_DIM


@functools.partial(
    pl.kernel,
    out_type=jax.ShapeDtypeStruct((L, NC_DIM, B), jnp.float32),
    mesh=_mesh,
    compiler_params=_params,
    scratch_types=[
        pltpu.VMEM((128,), jnp.float32),
        pltpu.VMEM((128,), jnp.float32),
        pltpu.VMEM((128,), jnp.int32),
        pltpu.VMEM((CATE_DIM, CATE_ROWS), jnp.float32),
        pltpu.VMEM((16, 16), jnp.float32),
        pltpu.VMEM((NC_DIM, 128), jnp.float32),
    ],
)
def _enc_nc(pT_h, cT_h, catT_h, ctT_h, wb_h, pa_h,
            pv, cv, gv, ctab, wbv, slab):
    wid = lax.axis_index("s") * 2 + lax.axis_index("c")

    pltpu.sync_copy(ctT_h, ctab)
    pltpu.sync_copy(wb_h, wbv)
    ws = [wbv[2 * d, :] for d in range(NUM_DIM)]
    bs = [wbv[2 * d + 1, :] for d in range(NUM_DIM)]

    def unit_body(u, carry):
        unit = u * NW + wid
        l = unit // NBLK
        b0 = (unit % NBLK) * 128

        pltpu.sync_copy(pT_h.at[l, pl.ds(b0, 128)], pv)
        pltpu.sync_copy(cT_h.at[l, pl.ds(b0, 128)], cv)
        pltpu.sync_copy(catT_h.at[l, pl.ds(b0, 128)], gv)

        def grp_a(g, c2):
            s = g * 16
            p = pv[pl.ds(s, 16)]
            c = cv[pl.ds(s, 16)]
            pm = p == p
            cm = c == c
            pc = jnp.where(pm, p, 0.0)
            cc = jnp.where(cm, c, 0.0)
            for d in range(NUM_DIM):
                vp = jnp.maximum(pc * ws[d] + bs[d], 0.0)
                slab[d, pl.ds(s, 16)] = jnp.where(pm, vp, 0.0)
                vc = jnp.maximum(cc * ws[d] + bs[d], 0.0)
                slab[NUM_DIM + d, pl.ds(s, 16)] = jnp.where(cm, vc, 0.0)
            ids = gv[pl.ds(s, 16)]
            for k in range(CATE_DIM):
                col = plsc.load_gather(
                    ctab, [jnp.full((16,), k, jnp.int32), ids])
                slab[2 * NUM_DIM + k, pl.ds(s, 16)] = col
            return c2

        lax.fori_loop(0, 8, grp_a, 0)
        pltpu.sync_copy(slab, pa_h.at[l, :, pl.ds(b0, 128)])
        return carry

    lax.fori_loop(0, UNITS, unit_body, 0)


@functools.partial(
    pl.kernel,
    out_type=jax.ShapeDtypeStruct((L, OUT_DIM, B), jnp.float32),
    mesh=_mesh,
    compiler_params=_params,
    scratch_types=[
        pltpu.VMEM((128,), jnp.int32),
        pltpu.VMEM((128, 128), jnp.float32),
        pltpu.VMEM((OUT_DIM, 128), jnp.float32),
        pltpu.SemaphoreType.DMA,
    ],
)
def _enc_item(itT_h, rt_h, pa_h, out_h, iv_, irows, slab, sem):
    wid = lax.axis_index("s") * 2 + lax.axis_index("c")

    def unit_body(u, carry):
        unit = u * NW + wid
        l = unit // NBLK
        b0 = (unit % NBLK) * 128

        pltpu.sync_copy(itT_h.at[l, pl.ds(b0, 128)], iv_)
        gather = pltpu.async_copy(rt_h.at[iv_], irows, sem)
        pltpu.sync_copy(pa_h.at[l, pl.ds(0, NUM_DIM * 2), pl.ds(b0, 128)],
                        slab.at[pl.ds(0, NUM_DIM * 2)])
        pltpu.sync_copy(pa_h.at[l, pl.ds(NUM_DIM * 2, CATE_DIM),
                                pl.ds(b0, 128)],
                        slab.at[pl.ds(2 * NUM_DIM + ITEM_DIM, CATE_DIM)])
        gather.wait()

        def grp_b(g, c2):
            s = g * 16
            iv16 = lax.iota(jnp.int32, 16) + s
            for c in range(ITEM_DIM):
                v = plsc.load_gather(
                    irows, [iv16, jnp.full((16,), c, jnp.int32)])
                slab[2 * NUM_DIM + c, pl.ds(s, 16)] = v
            return c2

        lax.fori_loop(0, 8, grp_b, 0)
        pltpu.sync_copy(slab, out_h.at[l, :, pl.ds(b0, 128)])
        return carry

    lax.fori_loop(0, UNITS, unit_body, 0)


def kernel(price, ctr, item_id, cate_id, W_num, b_num, table_items,
           table_cates):
    wb = jnp.stack([W_num[0], b_num], axis=1).reshape(16)
    wb = jnp.broadcast_to(wb[:, None], (16, 16))
    partial = _enc_nc(price.T, ctr.T, cate_id.T, table_cates.T, wb)
    rtab = _relayout(table_items)
    outT = _enc_item(item_id.T, rtab, partial)
    return outT.transpose(2, 0, 1)


# speedup vs baseline: 2.9919x; 1.0044x over previous
import functools

import jax
import jax.numpy as jnp
from jax import lax
from jax.experimental import pallas as pl
from jax.experimental.pallas import tpu as pltpu
from jax.experimental.pallas import tpu_sc as plsc

B, L = 16384, 50
NUM_DIM = 8
ITEM_DIM = 50
CATE_DIM = 12
OUT_DIM = 2 * NUM_DIM + ITEM_DIM + CATE_DIM
CATE_ROWS = 1001
ITEM_ROWS = 1000001
RT_ROWS = 1000064
FULL_COLS = ITEM_ROWS // 128
TAIL0 = FULL_COLS * 128

NW = 32
COL_UNITS = 245
NBLK = B // 128
UNITS = L * NBLK // NW

_mesh = plsc.VectorSubcoreMesh(core_axis_name="c", subcore_axis_name="s")
_params = pltpu.CompilerParams(needs_layout_passes=False)


RL_BLK = 4096


def _relayout_body(t_ref, rt_ref):
    rt_ref[...] = jnp.pad(t_ref[...], ((0, 0), (0, 128 - ITEM_DIM)))


def _relayout(table_items):
    grid = (RT_ROWS + RL_BLK - 1) // RL_BLK
    return pl.pallas_call(
        _relayout_body,
        grid=(grid,),
        in_specs=[pl.BlockSpec((RL_BLK, ITEM_DIM), lambda i: (i, 0))],
        out_specs=pl.BlockSpec((RL_BLK, 128), lambda i: (i, 0)),
        out_shape=jax.ShapeDtypeStruct((RT_ROWS, 128), jnp.float32),
    )(table_items)


NC_DIM = 2 * NUM_DIM + CATE_DIM


@functools.partial(
    pl.kernel,
    out_type=jax.ShapeDtypeStruct((L, NC_DIM, B), jnp.float32),
    mesh=_mesh,
    compiler_params=_params,
    scratch_types=[
        pltpu.VMEM((128,), jnp.float32),
        pltpu.VMEM((128,), jnp.float32),
        pltpu.VMEM((128,), jnp.int32),
        pltpu.VMEM((CATE_DIM, CATE_ROWS), jnp.float32),
        pltpu.VMEM((16, 16), jnp.float32),
        pltpu.VMEM((NC_DIM, 128), jnp.float32),
    ],
)
def _enc_nc(pT_h, cT_h, catT_h, ctT_h, wb_h, pa_h,
            pv, cv, gv, ctab, wbv, slab):
    wid = lax.axis_index("s") * 2 + lax.axis_index("c")

    pltpu.sync_copy(ctT_h, ctab)
    pltpu.sync_copy(wb_h, wbv)
    ws = [wbv[2 * d, :] for d in range(NUM_DIM)]
    bs = [wbv[2 * d + 1, :] for d in range(NUM_DIM)]

    def unit_body(u, carry):
        unit = u * NW + wid
        l = unit // NBLK
        b0 = (unit % NBLK) * 128

        pltpu.sync_copy(pT_h.at[l, pl.ds(b0, 128)], pv)
        pltpu.sync_copy(cT_h.at[l, pl.ds(b0, 128)], cv)
        pltpu.sync_copy(catT_h.at[l, pl.ds(b0, 128)], gv)

        def grp_a(g, c2):
            s = g * 16
            p = pv[pl.ds(s, 16)]
            c = cv[pl.ds(s, 16)]
            pm = p == p
            cm = c == c
            pc = jnp.where(pm, p, 0.0)
            cc = jnp.where(cm, c, 0.0)
            for d in range(NUM_DIM):
                vp = jnp.maximum(pc * ws[d] + bs[d], 0.0)
                slab[d, pl.ds(s, 16)] = jnp.where(pm, vp, 0.0)
                vc = jnp.maximum(cc * ws[d] + bs[d], 0.0)
                slab[NUM_DIM + d, pl.ds(s, 16)] = jnp.where(cm, vc, 0.0)
            ids = gv[pl.ds(s, 16)]
            for k in range(CATE_DIM):
                col = plsc.load_gather(
                    ctab, [jnp.full((16,), k, jnp.int32), ids])
                slab[2 * NUM_DIM + k, pl.ds(s, 16)] = col
            return c2

        lax.fori_loop(0, 8, grp_a, 0)
        pltpu.sync_copy(slab, pa_h.at[l, :, pl.ds(b0, 128)])
        return carry

    lax.fori_loop(0, UNITS, unit_body, 0)


@functools.partial(
    pl.kernel,
    out_type=jax.ShapeDtypeStruct((L, OUT_DIM, B), jnp.float32),
    mesh=_mesh,
    compiler_params=_params,
    scratch_types=[
        pltpu.VMEM((128,), jnp.int32),
        pltpu.VMEM((128,), jnp.int32),
        pltpu.VMEM((128, 128), jnp.float32),
        pltpu.VMEM((128, 128), jnp.float32),
        pltpu.VMEM((OUT_DIM, 128), jnp.float32),
        pltpu.SemaphoreType.DMA,
        pltpu.SemaphoreType.DMA,
        pltpu.SemaphoreType.DMA,
        pltpu.SemaphoreType.DMA,
    ],
)
def _enc_item(itT_h, rt_h, pa_h, out_h,
              iv0, iv1, ir0, ir1, slab, si0, si1, sg0, sg1):
    wid = lax.axis_index("s") * 2 + lax.axis_index("c")

    def lb(u):
        unit = u * NW + wid
        return unit // NBLK, (unit % NBLK) * 128

    def ids_src(u):
        l, b0 = lb(u)
        return itT_h.at[l, pl.ds(b0, 128)]

    def process(u, iv, ir, sg):
        l, b0 = lb(u)
        pltpu.sync_copy(pa_h.at[l, pl.ds(0, NUM_DIM * 2), pl.ds(b0, 128)],
                        slab.at[pl.ds(0, NUM_DIM * 2)])
        pltpu.sync_copy(pa_h.at[l, pl.ds(NUM_DIM * 2, CATE_DIM),
                                pl.ds(b0, 128)],
                        slab.at[pl.ds(2 * NUM_DIM + ITEM_DIM, CATE_DIM)])
        pltpu.make_async_copy(rt_h.at[iv], ir, sg).wait()

        def grp_b(g, c2):
            s = g * 16
            iv16 = lax.iota(jnp.int32, 16) + s
            for c in range(ITEM_DIM):
                v = plsc.load_gather(
                    ir, [iv16, jnp.full((16,), c, jnp.int32)])
                slab[2 * NUM_DIM + c, pl.ds(s, 16)] = v
            return c2

        lax.fori_loop(0, 8, grp_b, 0)
        pltpu.sync_copy(slab, out_h.at[l, :, pl.ds(b0, 128)])

    pltpu.sync_copy(ids_src(0), iv0)
    pltpu.async_copy(rt_h.at[iv0], ir0, sg0)
    pltpu.async_copy(ids_src(1), iv1, si1)

    def pair_body(t, carry):
        k0 = 2 * t
        k1 = k0 + 1
        pltpu.make_async_copy(ids_src(k1), iv1, si1).wait()
        pltpu.async_copy(rt_h.at[iv1], ir1, sg1)
        process(k0, iv0, ir0, sg0)

        @pl.when(k0 + 2 < UNITS)
        def _():
            pltpu.async_copy(ids_src(k0 + 2), iv0, si0)

        @pl.when(k0 + 2 < UNITS)
        def _():
            pltpu.make_async_copy(ids_src(k0 + 2), iv0, si0).wait()
            pltpu.async_copy(rt_h.at[iv0], ir0, sg0)

        process(k1, iv1, ir1, sg1)

        @pl.when(k1 + 2 < UNITS)
        def _():
            pltpu.async_copy(ids_src(k1 + 2), iv1, si1)

        return carry

    lax.fori_loop(0, UNITS // 2, pair_body, 0)


def kernel(price, ctr, item_id, cate_id, W_num, b_num, table_items,
           table_cates):
    wb = jnp.stack([W_num[0], b_num], axis=1).reshape(16)
    wb = jnp.broadcast_to(wb[:, None], (16, 16))
    partial = _enc_nc(price.T, ctr.T, cate_id.T, table_cates.T, wb)
    rtab = _relayout(table_items)
    outT = _enc_item(item_id.T, rtab, partial)
    return outT.transpose(2, 0, 1)


# speedup vs baseline: 3.5317x; 1.1804x over previous
import functools

import jax
import jax.numpy as jnp
from jax import lax
from jax.experimental import pallas as pl
from jax.experimental.pallas import tpu as pltpu
from jax.experimental.pallas import tpu_sc as plsc

B, L = 16384, 50
NUM_DIM = 8
ITEM_DIM = 50
CATE_DIM = 12
OUT_DIM = 2 * NUM_DIM + ITEM_DIM + CATE_DIM
CATE_ROWS = 1001
ITEM_ROWS = 1000001
RT_ROWS = 1000064
FULL_COLS = ITEM_ROWS // 128
TAIL0 = FULL_COLS * 128

NW = 32
COL_UNITS = 245
NBLK = B // 128
UNITS = L * NBLK // NW

_mesh = plsc.VectorSubcoreMesh(core_axis_name="c", subcore_axis_name="s")
_params = pltpu.CompilerParams(needs_layout_passes=False)


RL_BLK = 4096


def _relayout_body(t_ref, rt_ref):
    rt_ref[...] = jnp.pad(t_ref[...], ((0, 0), (0, 128 - ITEM_DIM)))


def _relayout(table_items):
    grid = (RT_ROWS + RL_BLK - 1) // RL_BLK
    return pl.pallas_call(
        _relayout_body,
        grid=(grid,),
        in_specs=[pl.BlockSpec((RL_BLK, ITEM_DIM), lambda i: (i, 0))],
        out_specs=pl.BlockSpec((RL_BLK, 128), lambda i: (i, 0)),
        out_shape=jax.ShapeDtypeStruct((RT_ROWS, 128), jnp.float32),
    )(table_items)


NC_DIM = 2 * NUM_DIM + CATE_DIM


@functools.partial(
    pl.kernel,
    out_type=jax.ShapeDtypeStruct((L, NC_DIM, B), jnp.float32),
    mesh=_mesh,
    compiler_params=_params,
    scratch_types=[
        pltpu.VMEM((128,), jnp.float32),
        pltpu.VMEM((128,), jnp.float32),
        pltpu.VMEM((128,), jnp.int32),
        pltpu.VMEM((CATE_DIM, CATE_ROWS), jnp.float32),
        pltpu.VMEM((16, 16), jnp.float32),
        pltpu.VMEM((NC_DIM, 128), jnp.float32),
    ],
)
def _enc_nc(pT_h, cT_h, catT_h, ctT_h, wb_h, pa_h,
            pv, cv, gv, ctab, wbv, slab):
    wid = lax.axis_index("s") * 2 + lax.axis_index("c")

    pltpu.sync_copy(ctT_h, ctab)
    pltpu.sync_copy(wb_h, wbv)
    ws = [wbv[2 * d, :] for d in range(NUM_DIM)]
    bs = [wbv[2 * d + 1, :] for d in range(NUM_DIM)]

    def unit_body(u, carry):
        unit = u * NW + wid
        l = unit // NBLK
        b0 = (unit % NBLK) * 128

        pltpu.sync_copy(pT_h.at[l, pl.ds(b0, 128)], pv)
        pltpu.sync_copy(cT_h.at[l, pl.ds(b0, 128)], cv)
        pltpu.sync_copy(catT_h.at[l, pl.ds(b0, 128)], gv)

        def grp_a(g, c2):
            s = g * 16
            p = pv[pl.ds(s, 16)]
            c = cv[pl.ds(s, 16)]
            pm = p == p
            cm = c == c
            pc = jnp.where(pm, p, 0.0)
            cc = jnp.where(cm, c, 0.0)
            for d in range(NUM_DIM):
                vp = jnp.maximum(pc * ws[d] + bs[d], 0.0)
                slab[d, pl.ds(s, 16)] = jnp.where(pm, vp, 0.0)
                vc = jnp.maximum(cc * ws[d] + bs[d], 0.0)
                slab[NUM_DIM + d, pl.ds(s, 16)] = jnp.where(cm, vc, 0.0)
            ids = gv[pl.ds(s, 16)]
            for k in range(CATE_DIM):
                col = plsc.load_gather(
                    ctab, [jnp.full((16,), k, jnp.int32), ids])
                slab[2 * NUM_DIM + k, pl.ds(s, 16)] = col
            return c2

        lax.fori_loop(0, 8, grp_a, 0)
        pltpu.sync_copy(slab, pa_h.at[l, :, pl.ds(b0, 128)])
        return carry

    lax.fori_loop(0, UNITS, unit_body, 0)


@functools.partial(
    pl.kernel,
    out_type=jax.ShapeDtypeStruct((L, OUT_DIM, B), jnp.float32),
    mesh=_mesh,
    compiler_params=_params,
    scratch_types=[
        pltpu.VMEM((128,), jnp.int32),
        pltpu.VMEM((128,), jnp.int32),
        pltpu.VMEM((128, 128), jnp.float32),
        pltpu.VMEM((128, 128), jnp.float32),
        pltpu.VMEM((OUT_DIM, 128), jnp.float32),
        pltpu.SemaphoreType.DMA,
        pltpu.SemaphoreType.DMA,
        pltpu.SemaphoreType.DMA,
        pltpu.SemaphoreType.DMA,
    ],
)
def _enc_item(itT_h, rt_h, pa_h, out_h,
              iv0, iv1, ir0, ir1, slab, si0, si1, sg0, sg1):
    wid = lax.axis_index("s") * 2 + lax.axis_index("c")

    def lb(u):
        unit = u * NW + wid
        return unit // NBLK, (unit % NBLK) * 128

    def ids_src(u):
        l, b0 = lb(u)
        return itT_h.at[l, pl.ds(b0, 128)]

    def process(u, iv, ir, sg):
        l, b0 = lb(u)
        pltpu.sync_copy(pa_h.at[l, pl.ds(0, NUM_DIM * 2), pl.ds(b0, 128)],
                        slab.at[pl.ds(0, NUM_DIM * 2)])
        pltpu.sync_copy(pa_h.at[l, pl.ds(NUM_DIM * 2, CATE_DIM),
                                pl.ds(b0, 128)],
                        slab.at[pl.ds(2 * NUM_DIM + ITEM_DIM, CATE_DIM)])
        pltpu.make_async_copy(rt_h.at[iv], ir, sg).wait()

        def grp_b(g, c2):
            s = g * 16
            iv16 = lax.iota(jnp.int32, 16) + s
            for cb in range(0, ITEM_DIM, 4):
                n = min(4, ITEM_DIM - cb)
                vs = [plsc.load_gather(
                    ir, [iv16, jnp.full((16,), cb + j, jnp.int32)])
                    for j in range(n)]
                for j in range(n):
                    slab[2 * NUM_DIM + cb + j, pl.ds(s, 16)] = vs[j]
            return c2

        lax.fori_loop(0, 8, grp_b, 0)
        pltpu.sync_copy(slab, out_h.at[l, :, pl.ds(b0, 128)])

    pltpu.sync_copy(ids_src(0), iv0)
    pltpu.async_copy(rt_h.at[iv0], ir0, sg0)
    pltpu.async_copy(ids_src(1), iv1, si1)

    def pair_body(t, carry):
        k0 = 2 * t
        k1 = k0 + 1
        pltpu.make_async_copy(ids_src(k1), iv1, si1).wait()
        pltpu.async_copy(rt_h.at[iv1], ir1, sg1)
        process(k0, iv0, ir0, sg0)

        @pl.when(k0 + 2 < UNITS)
        def _():
            pltpu.async_copy(ids_src(k0 + 2), iv0, si0)

        @pl.when(k0 + 2 < UNITS)
        def _():
            pltpu.make_async_copy(ids_src(k0 + 2), iv0, si0).wait()
            pltpu.async_copy(rt_h.at[iv0], ir0, sg0)

        process(k1, iv1, ir1, sg1)

        @pl.when(k1 + 2 < UNITS)
        def _():
            pltpu.async_copy(ids_src(k1 + 2), iv1, si1)

        return carry

    lax.fori_loop(0, UNITS // 2, pair_body, 0)


def kernel(price, ctr, item_id, cate_id, W_num, b_num, table_items,
           table_cates):
    wb = jnp.stack([W_num[0], b_num], axis=1).reshape(16)
    wb = jnp.broadcast_to(wb[:, None], (16, 16))
    partial = _enc_nc(price.T, ctr.T, cate_id.T, table_cates.T, wb)
    rtab = _relayout(table_items)
    outT = _enc_item(item_id.T, rtab, partial)
    return outT.transpose(2, 0, 1)


# speedup vs baseline: 3.6237x; 1.0260x over previous
import functools

import jax
import jax.numpy as jnp
from jax import lax
from jax.experimental import pallas as pl
from jax.experimental.pallas import tpu as pltpu
from jax.experimental.pallas import tpu_sc as plsc

B, L = 16384, 50
NUM_DIM = 8
ITEM_DIM = 50
CATE_DIM = 12
OUT_DIM = 2 * NUM_DIM + ITEM_DIM + CATE_DIM
CATE_ROWS = 1001
ITEM_ROWS = 1000001
RT_ROWS = 1000064
FULL_COLS = ITEM_ROWS // 128
TAIL0 = FULL_COLS * 128

NW = 32
COL_UNITS = 245
NBLK = B // 128
UNITS = L * NBLK // NW

_mesh = plsc.VectorSubcoreMesh(core_axis_name="c", subcore_axis_name="s")
_params = pltpu.CompilerParams(needs_layout_passes=False)


RL_BLK = 4096


def _relayout_body(t_ref, rt_ref):
    rt_ref[...] = jnp.pad(t_ref[...], ((0, 0), (0, 128 - ITEM_DIM)))


def _relayout(table_items):
    grid = (RT_ROWS + RL_BLK - 1) // RL_BLK
    return pl.pallas_call(
        _relayout_body,
        grid=(grid,),
        in_specs=[pl.BlockSpec((RL_BLK, ITEM_DIM), lambda i: (i, 0))],
        out_specs=pl.BlockSpec((RL_BLK, 128), lambda i: (i, 0)),
        out_shape=jax.ShapeDtypeStruct((RT_ROWS, 128), jnp.float32),
    )(table_items)


NC_DIM = 2 * NUM_DIM + CATE_DIM


@functools.partial(
    pl.kernel,
    out_type=jax.ShapeDtypeStruct((L, NC_DIM, B), jnp.float32),
    mesh=_mesh,
    compiler_params=_params,
    scratch_types=[
        pltpu.VMEM((128,), jnp.float32),
        pltpu.VMEM((128,), jnp.float32),
        pltpu.VMEM((128,), jnp.int32),
        pltpu.VMEM((CATE_DIM, CATE_ROWS), jnp.float32),
        pltpu.VMEM((16, 16), jnp.float32),
        pltpu.VMEM((NC_DIM, 128), jnp.float32),
    ],
)
def _enc_nc(pT_h, cT_h, catT_h, ctT_h, wb_h, pa_h,
            pv, cv, gv, ctab, wbv, slab):
    wid = lax.axis_index("s") * 2 + lax.axis_index("c")

    pltpu.sync_copy(ctT_h, ctab)
    pltpu.sync_copy(wb_h, wbv)
    ws = [wbv[2 * d, :] for d in range(NUM_DIM)]
    bs = [wbv[2 * d + 1, :] for d in range(NUM_DIM)]

    def unit_body(u, carry):
        unit = u * NW + wid
        l = unit // NBLK
        b0 = (unit % NBLK) * 128

        pltpu.sync_copy(pT_h.at[l, pl.ds(b0, 128)], pv)
        pltpu.sync_copy(cT_h.at[l, pl.ds(b0, 128)], cv)
        pltpu.sync_copy(catT_h.at[l, pl.ds(b0, 128)], gv)

        def grp_a(g, c2):
            s = g * 16
            p = pv[pl.ds(s, 16)]
            c = cv[pl.ds(s, 16)]
            pm = p == p
            cm = c == c
            pc = jnp.where(pm, p, 0.0)
            cc = jnp.where(cm, c, 0.0)
            for d in range(NUM_DIM):
                vp = jnp.maximum(pc * ws[d] + bs[d], 0.0)
                slab[d, pl.ds(s, 16)] = jnp.where(pm, vp, 0.0)
                vc = jnp.maximum(cc * ws[d] + bs[d], 0.0)
                slab[NUM_DIM + d, pl.ds(s, 16)] = jnp.where(cm, vc, 0.0)
            ids = gv[pl.ds(s, 16)]
            cols = [plsc.load_gather(
                ctab, [jnp.full((16,), k, jnp.int32), ids])
                for k in range(CATE_DIM)]
            for k in range(CATE_DIM):
                slab[2 * NUM_DIM + k, pl.ds(s, 16)] = cols[k]
            return c2

        lax.fori_loop(0, 8, grp_a, 0)
        pltpu.sync_copy(slab, pa_h.at[l, :, pl.ds(b0, 128)])
        return carry

    lax.fori_loop(0, UNITS, unit_body, 0)


@functools.partial(
    pl.kernel,
    out_type=jax.ShapeDtypeStruct((L, OUT_DIM, B), jnp.float32),
    mesh=_mesh,
    compiler_params=_params,
    scratch_types=[
        pltpu.VMEM((128,), jnp.int32),
        pltpu.VMEM((128,), jnp.int32),
        pltpu.VMEM((128, 128), jnp.float32),
        pltpu.VMEM((128, 128), jnp.float32),
        pltpu.VMEM((OUT_DIM, 128), jnp.float32),
        pltpu.SemaphoreType.DMA,
        pltpu.SemaphoreType.DMA,
        pltpu.SemaphoreType.DMA,
        pltpu.SemaphoreType.DMA,
    ],
)
def _enc_item(itT_h, rt_h, pa_h, out_h,
              iv0, iv1, ir0, ir1, slab, si0, si1, sg0, sg1):
    wid = lax.axis_index("s") * 2 + lax.axis_index("c")

    def lb(u):
        unit = u * NW + wid
        return unit // NBLK, (unit % NBLK) * 128

    def ids_src(u):
        l, b0 = lb(u)
        return itT_h.at[l, pl.ds(b0, 128)]

    def process(u, iv, ir, sg):
        l, b0 = lb(u)
        pltpu.sync_copy(pa_h.at[l, pl.ds(0, NUM_DIM * 2), pl.ds(b0, 128)],
                        slab.at[pl.ds(0, NUM_DIM * 2)])
        pltpu.sync_copy(pa_h.at[l, pl.ds(NUM_DIM * 2, CATE_DIM),
                                pl.ds(b0, 128)],
                        slab.at[pl.ds(2 * NUM_DIM + ITEM_DIM, CATE_DIM)])
        pltpu.make_async_copy(rt_h.at[iv], ir, sg).wait()

        def grp_b(g, c2):
            s = g * 16
            iv16 = lax.iota(jnp.int32, 16) + s
            for cb in range(0, ITEM_DIM, 8):
                n = min(8, ITEM_DIM - cb)
                vs = [plsc.load_gather(
                    ir, [iv16, jnp.full((16,), cb + j, jnp.int32)])
                    for j in range(n)]
                for j in range(n):
                    slab[2 * NUM_DIM + cb + j, pl.ds(s, 16)] = vs[j]
            return c2

        lax.fori_loop(0, 8, grp_b, 0)
        pltpu.sync_copy(slab, out_h.at[l, :, pl.ds(b0, 128)])

    pltpu.sync_copy(ids_src(0), iv0)
    pltpu.async_copy(rt_h.at[iv0], ir0, sg0)
    pltpu.async_copy(ids_src(1), iv1, si1)

    def pair_body(t, carry):
        k0 = 2 * t
        k1 = k0 + 1
        pltpu.make_async_copy(ids_src(k1), iv1, si1).wait()
        pltpu.async_copy(rt_h.at[iv1], ir1, sg1)
        process(k0, iv0, ir0, sg0)

        @pl.when(k0 + 2 < UNITS)
        def _():
            pltpu.async_copy(ids_src(k0 + 2), iv0, si0)

        @pl.when(k0 + 2 < UNITS)
        def _():
            pltpu.make_async_copy(ids_src(k0 + 2), iv0, si0).wait()
            pltpu.async_copy(rt_h.at[iv0], ir0, sg0)

        process(k1, iv1, ir1, sg1)

        @pl.when(k1 + 2 < UNITS)
        def _():
            pltpu.async_copy(ids_src(k1 + 2), iv1, si1)

        return carry

    lax.fori_loop(0, UNITS // 2, pair_body, 0)


def kernel(price, ctr, item_id, cate_id, W_num, b_num, table_items,
           table_cates):
    wb = jnp.stack([W_num[0], b_num], axis=1).reshape(16)
    wb = jnp.broadcast_to(wb[:, None], (16, 16))
    partial = _enc_nc(price.T, ctr.T, cate_id.T, table_cates.T, wb)
    rtab = _relayout(table_items)
    outT = _enc_item(item_id.T, rtab, partial)
    return outT.transpose(2, 0, 1)


# speedup vs baseline: 3.6266x; 1.0008x over previous
import functools

import jax
import jax.numpy as jnp
from jax import lax
from jax.experimental import pallas as pl
from jax.experimental.pallas import tpu as pltpu
from jax.experimental.pallas import tpu_sc as plsc

B, L = 16384, 50
NUM_DIM = 8
ITEM_DIM = 50
CATE_DIM = 12
OUT_DIM = 2 * NUM_DIM + ITEM_DIM + CATE_DIM
CATE_ROWS = 1001
ITEM_ROWS = 1000001
RT_ROWS = 1000064
FULL_COLS = ITEM_ROWS // 128
TAIL0 = FULL_COLS * 128

NW = 32
COL_UNITS = 245
NBLK = B // 128
UNITS = L * NBLK // NW

_mesh = plsc.VectorSubcoreMesh(core_axis_name="c", subcore_axis_name="s")
_params = pltpu.CompilerParams(needs_layout_passes=False)


RL_BLK = 4096


def _relayout_body(t_ref, rt_ref):
    rt_ref[...] = jnp.pad(t_ref[...], ((0, 0), (0, 128 - ITEM_DIM)))


def _relayout(table_items):
    grid = (RT_ROWS + RL_BLK - 1) // RL_BLK
    return pl.pallas_call(
        _relayout_body,
        grid=(grid,),
        in_specs=[pl.BlockSpec((RL_BLK, ITEM_DIM), lambda i: (i, 0))],
        out_specs=pl.BlockSpec((RL_BLK, 128), lambda i: (i, 0)),
        out_shape=jax.ShapeDtypeStruct((RT_ROWS, 128), jnp.float32),
        compiler_params=pltpu.CompilerParams(
            dimension_semantics=("parallel",)),
    )(table_items)


NC_DIM = 2 * NUM_DIM + CATE_DIM


@functools.partial(
    pl.kernel,
    out_type=jax.ShapeDtypeStruct((L, NC_DIM, B), jnp.float32),
    mesh=_mesh,
    compiler_params=_params,
    scratch_types=[
        pltpu.VMEM((128,), jnp.float32),
        pltpu.VMEM((128,), jnp.float32),
        pltpu.VMEM((128,), jnp.int32),
        pltpu.VMEM((CATE_DIM, CATE_ROWS), jnp.float32),
        pltpu.VMEM((16, 16), jnp.float32),
        pltpu.VMEM((NC_DIM, 128), jnp.float32),
    ],
)
def _enc_nc(pT_h, cT_h, catT_h, ctT_h, wb_h, pa_h,
            pv, cv, gv, ctab, wbv, slab):
    wid = lax.axis_index("s") * 2 + lax.axis_index("c")

    pltpu.sync_copy(ctT_h, ctab)
    pltpu.sync_copy(wb_h, wbv)
    ws = [wbv[2 * d, :] for d in range(NUM_DIM)]
    bs = [wbv[2 * d + 1, :] for d in range(NUM_DIM)]

    def unit_body(u, carry):
        unit = u * NW + wid
        l = unit // NBLK
        b0 = (unit % NBLK) * 128

        pltpu.sync_copy(pT_h.at[l, pl.ds(b0, 128)], pv)
        pltpu.sync_copy(cT_h.at[l, pl.ds(b0, 128)], cv)
        pltpu.sync_copy(catT_h.at[l, pl.ds(b0, 128)], gv)

        def grp_a(g, c2):
            s = g * 16
            p = pv[pl.ds(s, 16)]
            c = cv[pl.ds(s, 16)]
            pm = p == p
            cm = c == c
            pc = jnp.where(pm, p, 0.0)
            cc = jnp.where(cm, c, 0.0)
            for d in range(NUM_DIM):
                vp = jnp.maximum(pc * ws[d] + bs[d], 0.0)
                slab[d, pl.ds(s, 16)] = jnp.where(pm, vp, 0.0)
                vc = jnp.maximum(cc * ws[d] + bs[d], 0.0)
                slab[NUM_DIM + d, pl.ds(s, 16)] = jnp.where(cm, vc, 0.0)
            ids = gv[pl.ds(s, 16)]
            cols = [plsc.load_gather(
                ctab, [jnp.full((16,), k, jnp.int32), ids])
                for k in range(CATE_DIM)]
            for k in range(CATE_DIM):
                slab[2 * NUM_DIM + k, pl.ds(s, 16)] = cols[k]
            return c2

        lax.fori_loop(0, 8, grp_a, 0)
        pltpu.sync_copy(slab, pa_h.at[l, :, pl.ds(b0, 128)])
        return carry

    lax.fori_loop(0, UNITS, unit_body, 0)


@functools.partial(
    pl.kernel,
    out_type=jax.ShapeDtypeStruct((L, OUT_DIM, B), jnp.float32),
    mesh=_mesh,
    compiler_params=_params,
    scratch_types=[
        pltpu.VMEM((128,), jnp.int32),
        pltpu.VMEM((128,), jnp.int32),
        pltpu.VMEM((128, 128), jnp.float32),
        pltpu.VMEM((128, 128), jnp.float32),
        pltpu.VMEM((OUT_DIM, 128), jnp.float32),
        pltpu.SemaphoreType.DMA,
        pltpu.SemaphoreType.DMA,
        pltpu.SemaphoreType.DMA,
        pltpu.SemaphoreType.DMA,
    ],
)
def _enc_item(itT_h, rt_h, pa_h, out_h,
              iv0, iv1, ir0, ir1, slab, si0, si1, sg0, sg1):
    wid = lax.axis_index("s") * 2 + lax.axis_index("c")

    def lb(u):
        unit = u * NW + wid
        return unit // NBLK, (unit % NBLK) * 128

    def ids_src(u):
        l, b0 = lb(u)
        return itT_h.at[l, pl.ds(b0, 128)]

    def process(u, iv, ir, sg):
        l, b0 = lb(u)
        pltpu.sync_copy(pa_h.at[l, pl.ds(0, NUM_DIM * 2), pl.ds(b0, 128)],
                        slab.at[pl.ds(0, NUM_DIM * 2)])
        pltpu.sync_copy(pa_h.at[l, pl.ds(NUM_DIM * 2, CATE_DIM),
                                pl.ds(b0, 128)],
                        slab.at[pl.ds(2 * NUM_DIM + ITEM_DIM, CATE_DIM)])
        pltpu.make_async_copy(rt_h.at[iv], ir, sg).wait()

        def grp_b(g, c2):
            s = g * 16
            iv16 = lax.iota(jnp.int32, 16) + s
            for cb in range(0, ITEM_DIM, 8):
                n = min(8, ITEM_DIM - cb)
                vs = [plsc.load_gather(
                    ir, [iv16, jnp.full((16,), cb + j, jnp.int32)])
                    for j in range(n)]
                for j in range(n):
                    slab[2 * NUM_DIM + cb + j, pl.ds(s, 16)] = vs[j]
            return c2

        lax.fori_loop(0, 8, grp_b, 0)
        pltpu.sync_copy(slab, out_h.at[l, :, pl.ds(b0, 128)])

    pltpu.sync_copy(ids_src(0), iv0)
    pltpu.async_copy(rt_h.at[iv0], ir0, sg0)
    pltpu.async_copy(ids_src(1), iv1, si1)

    def pair_body(t, carry):
        k0 = 2 * t
        k1 = k0 + 1
        pltpu.make_async_copy(ids_src(k1), iv1, si1).wait()
        pltpu.async_copy(rt_h.at[iv1], ir1, sg1)
        process(k0, iv0, ir0, sg0)

        @pl.when(k0 + 2 < UNITS)
        def _():
            pltpu.async_copy(ids_src(k0 + 2), iv0, si0)

        @pl.when(k0 + 2 < UNITS)
        def _():
            pltpu.make_async_copy(ids_src(k0 + 2), iv0, si0).wait()
            pltpu.async_copy(rt_h.at[iv0], ir0, sg0)

        process(k1, iv1, ir1, sg1)

        @pl.when(k1 + 2 < UNITS)
        def _():
            pltpu.async_copy(ids_src(k1 + 2), iv1, si1)

        return carry

    lax.fori_loop(0, UNITS // 2, pair_body, 0)


def kernel(price, ctr, item_id, cate_id, W_num, b_num, table_items,
           table_cates):
    wb = jnp.stack([W_num[0], b_num], axis=1).reshape(16)
    wb = jnp.broadcast_to(wb[:, None], (16, 16))
    partial = _enc_nc(price.T, ctr.T, cate_id.T, table_cates.T, wb)
    rtab = _relayout(table_items)
    outT = _enc_item(item_id.T, rtab, partial)
    return outT.transpose(2, 0, 1)


# speedup vs baseline: 3.8250x; 1.0547x over previous
import functools

import jax
import jax.numpy as jnp
from jax import lax
from jax.experimental import pallas as pl
from jax.experimental.pallas import tpu as pltpu
from jax.experimental.pallas import tpu_sc as plsc

B, L = 16384, 50
NUM_DIM = 8
ITEM_DIM = 50
CATE_DIM = 12
OUT_DIM = 2 * NUM_DIM + ITEM_DIM + CATE_DIM
CATE_ROWS = 1001
ITEM_ROWS = 1000001
RT_ROWS = 1000064
FULL_COLS = ITEM_ROWS // 128
TAIL0 = FULL_COLS * 128

NW = 32
COL_UNITS = 245
NBLK = B // 128
UNITS = L * NBLK // NW

_mesh = plsc.VectorSubcoreMesh(core_axis_name="c", subcore_axis_name="s")
_params = pltpu.CompilerParams(needs_layout_passes=False)


RL_BLK = 4096
SPLIT = 503808
RL_GRID = SPLIT // RL_BLK
HI_LAST = (ITEM_ROWS - 1) // RL_BLK


def _relayout_body(lo_ref, hi_ref, rt_ref):
    pad = jnp.zeros((RL_BLK, 64 - ITEM_DIM), jnp.float32)
    rt_ref[...] = jnp.concatenate(
        [lo_ref[...], pad, hi_ref[...], pad], axis=1)


def _relayout(table_items):
    return pl.pallas_call(
        _relayout_body,
        grid=(RL_GRID,),
        in_specs=[
            pl.BlockSpec((RL_BLK, ITEM_DIM), lambda i: (i, 0)),
            pl.BlockSpec((RL_BLK, ITEM_DIM),
                         lambda i: (jnp.minimum(i + RL_GRID, HI_LAST), 0)),
        ],
        out_specs=pl.BlockSpec((RL_BLK, 128), lambda i: (i, 0)),
        out_shape=jax.ShapeDtypeStruct((SPLIT, 128), jnp.float32),
        compiler_params=pltpu.CompilerParams(
            dimension_semantics=("parallel",)),
    )(table_items, table_items)


NC_DIM = 2 * NUM_DIM + CATE_DIM


@functools.partial(
    pl.kernel,
    out_type=jax.ShapeDtypeStruct((L, NC_DIM, B), jnp.float32),
    mesh=_mesh,
    compiler_params=_params,
    scratch_types=[
        pltpu.VMEM((128,), jnp.float32),
        pltpu.VMEM((128,), jnp.float32),
        pltpu.VMEM((128,), jnp.int32),
        pltpu.VMEM((CATE_DIM, CATE_ROWS), jnp.float32),
        pltpu.VMEM((16, 16), jnp.float32),
        pltpu.VMEM((NC_DIM, 128), jnp.float32),
    ],
)
def _enc_nc(pT_h, cT_h, catT_h, ctT_h, wb_h, pa_h,
            pv, cv, gv, ctab, wbv, slab):
    wid = lax.axis_index("s") * 2 + lax.axis_index("c")

    pltpu.sync_copy(ctT_h, ctab)
    pltpu.sync_copy(wb_h, wbv)
    ws = [wbv[2 * d, :] for d in range(NUM_DIM)]
    bs = [wbv[2 * d + 1, :] for d in range(NUM_DIM)]

    def unit_body(u, carry):
        unit = u * NW + wid
        l = unit // NBLK
        b0 = (unit % NBLK) * 128

        pltpu.sync_copy(pT_h.at[l, pl.ds(b0, 128)], pv)
        pltpu.sync_copy(cT_h.at[l, pl.ds(b0, 128)], cv)
        pltpu.sync_copy(catT_h.at[l, pl.ds(b0, 128)], gv)

        def grp_a(g, c2):
            s = g * 16
            p = pv[pl.ds(s, 16)]
            c = cv[pl.ds(s, 16)]
            pm = p == p
            cm = c == c
            pc = jnp.where(pm, p, 0.0)
            cc = jnp.where(cm, c, 0.0)
            for d in range(NUM_DIM):
                vp = jnp.maximum(pc * ws[d] + bs[d], 0.0)
                slab[d, pl.ds(s, 16)] = jnp.where(pm, vp, 0.0)
                vc = jnp.maximum(cc * ws[d] + bs[d], 0.0)
                slab[NUM_DIM + d, pl.ds(s, 16)] = jnp.where(cm, vc, 0.0)
            ids = gv[pl.ds(s, 16)]
            cols = [plsc.load_gather(
                ctab, [jnp.full((16,), k, jnp.int32), ids])
                for k in range(CATE_DIM)]
            for k in range(CATE_DIM):
                slab[2 * NUM_DIM + k, pl.ds(s, 16)] = cols[k]
            return c2

        lax.fori_loop(0, 8, grp_a, 0)
        pltpu.sync_copy(slab, pa_h.at[l, :, pl.ds(b0, 128)])
        return carry

    lax.fori_loop(0, UNITS, unit_body, 0)


@functools.partial(
    pl.kernel,
    out_type=jax.ShapeDtypeStruct((L, OUT_DIM, B), jnp.float32),
    mesh=_mesh,
    compiler_params=_params,
    scratch_types=[
        pltpu.VMEM((128,), jnp.int32),
        pltpu.VMEM((128,), jnp.int32),
        pltpu.VMEM((128,), jnp.int32),
        pltpu.VMEM((128,), jnp.int32),
        pltpu.VMEM((128, 128), jnp.float32),
        pltpu.VMEM((128, 128), jnp.float32),
        pltpu.VMEM((OUT_DIM, 128), jnp.float32),
        pltpu.SemaphoreType.DMA,
        pltpu.SemaphoreType.DMA,
        pltpu.SemaphoreType.DMA,
        pltpu.SemaphoreType.DMA,
    ],
)
def _enc_item(itT_h, rt_h, pa_h, out_h,
              iv0, iv1, r0, r1, ir0, ir1, slab, si0, si1, sg0, sg1):
    wid = lax.axis_index("s") * 2 + lax.axis_index("c")

    def lb(u):
        unit = u * NW + wid
        return unit // NBLK, (unit % NBLK) * 128

    def ids_src(u):
        l, b0 = lb(u)
        return itT_h.at[l, pl.ds(b0, 128)]

    def adjust(ivr, radj):
        def g8(g, c2):
            s = g * 16
            v = ivr[pl.ds(s, 16)]
            radj[pl.ds(s, 16)] = v - jnp.where(v >= SPLIT, SPLIT, 0)
            return c2

        lax.fori_loop(0, 8, g8, 0)

    def process(u, ivraw, r, ir, sg):
        l, b0 = lb(u)
        pltpu.sync_copy(pa_h.at[l, pl.ds(0, NUM_DIM * 2), pl.ds(b0, 128)],
                        slab.at[pl.ds(0, NUM_DIM * 2)])
        pltpu.sync_copy(pa_h.at[l, pl.ds(NUM_DIM * 2, CATE_DIM),
                                pl.ds(b0, 128)],
                        slab.at[pl.ds(2 * NUM_DIM + ITEM_DIM, CATE_DIM)])
        pltpu.make_async_copy(rt_h.at[r], ir, sg).wait()

        def grp_b(g, c2):
            s = g * 16
            iv16 = lax.iota(jnp.int32, 16) + s
            base16 = jnp.where(ivraw[pl.ds(s, 16)] >= SPLIT, 64, 0)
            for cb in range(0, ITEM_DIM, 8):
                n = min(8, ITEM_DIM - cb)
                vs = [plsc.load_gather(ir, [iv16, base16 + (cb + j)])
                      for j in range(n)]
                for j in range(n):
                    slab[2 * NUM_DIM + cb + j, pl.ds(s, 16)] = vs[j]
            return c2

        lax.fori_loop(0, 8, grp_b, 0)
        pltpu.sync_copy(slab, out_h.at[l, :, pl.ds(b0, 128)])

    pltpu.sync_copy(ids_src(0), iv0)
    adjust(iv0, r0)
    pltpu.async_copy(rt_h.at[r0], ir0, sg0)
    pltpu.async_copy(ids_src(1), iv1, si1)

    def pair_body(t, carry):
        k0 = 2 * t
        k1 = k0 + 1
        pltpu.make_async_copy(ids_src(k1), iv1, si1).wait()
        adjust(iv1, r1)
        pltpu.async_copy(rt_h.at[r1], ir1, sg1)
        process(k0, iv0, r0, ir0, sg0)

        @pl.when(k0 + 2 < UNITS)
        def _():
            pltpu.async_copy(ids_src(k0 + 2), iv0, si0)

        @pl.when(k0 + 2 < UNITS)
        def _():
            pltpu.make_async_copy(ids_src(k0 + 2), iv0, si0).wait()
            adjust(iv0, r0)
            pltpu.async_copy(rt_h.at[r0], ir0, sg0)

        process(k1, iv1, r1, ir1, sg1)

        @pl.when(k1 + 2 < UNITS)
        def _():
            pltpu.async_copy(ids_src(k1 + 2), iv1, si1)

        return carry

    lax.fori_loop(0, UNITS // 2, pair_body, 0)


def kernel(price, ctr, item_id, cate_id, W_num, b_num, table_items,
           table_cates):
    wb = jnp.stack([W_num[0], b_num], axis=1).reshape(16)
    wb = jnp.broadcast_to(wb[:, None], (16, 16))
    partial = _enc_nc(price.T, ctr.T, cate_id.T, table_cates.T, wb)
    rtab = _relayout(table_items)
    outT = _enc_item(item_id.T, rtab, partial)
    return outT.transpose(2, 0, 1)


# speedup vs baseline: 4.6265x; 1.2095x over previous
import functools

import jax
import jax.numpy as jnp
from jax import lax
from jax.experimental import pallas as pl
from jax.experimental.pallas import tpu as pltpu
from jax.experimental.pallas import tpu_sc as plsc

B, L = 16384, 50
NUM_DIM = 8
ITEM_DIM = 50
CATE_DIM = 12
OUT_DIM = 2 * NUM_DIM + ITEM_DIM + CATE_DIM
CATE_ROWS = 1001
ITEM_ROWS = 1000001
RT_ROWS = 1000064
FULL_COLS = ITEM_ROWS // 128
TAIL0 = FULL_COLS * 128

NW = 32
COL_UNITS = 245
NBLK = B // 128
UNITS = L * NBLK // NW

_mesh = plsc.VectorSubcoreMesh(core_axis_name="c", subcore_axis_name="s")
_params = pltpu.CompilerParams(needs_layout_passes=False)


RL_BLK = 4096
SPLIT = 503808
RL_GRID = SPLIT // RL_BLK
HI_LAST = (ITEM_ROWS - 1) // RL_BLK


def _relayout_body(lo_ref, hi_ref, rt_ref):
    pad = jnp.zeros((RL_BLK, 64 - ITEM_DIM), jnp.float32)
    rt_ref[...] = jnp.concatenate(
        [lo_ref[...], pad, hi_ref[...], pad], axis=1)


def _relayout(table_items):
    return pl.pallas_call(
        _relayout_body,
        grid=(RL_GRID,),
        in_specs=[
            pl.BlockSpec((RL_BLK, ITEM_DIM), lambda i: (i, 0)),
            pl.BlockSpec((RL_BLK, ITEM_DIM),
                         lambda i: (jnp.minimum(i + RL_GRID, HI_LAST), 0)),
        ],
        out_specs=pl.BlockSpec((RL_BLK, 128), lambda i: (i, 0)),
        out_shape=jax.ShapeDtypeStruct((SPLIT, 128), jnp.float32),
        compiler_params=pltpu.CompilerParams(
            dimension_semantics=("parallel",)),
    )(table_items, table_items)


NC_DIM = 2 * NUM_DIM + CATE_DIM


@functools.partial(
    pl.kernel,
    out_type=jax.ShapeDtypeStruct((L, NC_DIM, B), jnp.float32),
    mesh=_mesh,
    compiler_params=_params,
    scratch_types=[
        pltpu.VMEM((128,), jnp.float32),
        pltpu.VMEM((128,), jnp.float32),
        pltpu.VMEM((128,), jnp.int32),
        pltpu.VMEM((CATE_DIM, CATE_ROWS), jnp.float32),
        pltpu.VMEM((16, 16), jnp.float32),
        pltpu.VMEM((NC_DIM, 128), jnp.float32),
    ],
)
def _enc_nc(pT_h, cT_h, catT_h, ctT_h, wb_h, pa_h,
            pv, cv, gv, ctab, wbv, slab):
    wid = lax.axis_index("s") * 2 + lax.axis_index("c")

    pltpu.sync_copy(ctT_h, ctab)
    pltpu.sync_copy(wb_h, wbv)
    ws = [wbv[2 * d, :] for d in range(NUM_DIM)]
    bs = [wbv[2 * d + 1, :] for d in range(NUM_DIM)]

    def unit_body(u, carry):
        unit = u * NW + wid
        l = unit // NBLK
        b0 = (unit % NBLK) * 128

        pltpu.sync_copy(pT_h.at[l, pl.ds(b0, 128)], pv)
        pltpu.sync_copy(cT_h.at[l, pl.ds(b0, 128)], cv)
        pltpu.sync_copy(catT_h.at[l, pl.ds(b0, 128)], gv)

        def grp_a(g, c2):
            s = g * 16
            p = pv[pl.ds(s, 16)]
            c = cv[pl.ds(s, 16)]
            pm = p == p
            cm = c == c
            pc = jnp.where(pm, p, 0.0)
            cc = jnp.where(cm, c, 0.0)
            for d in range(NUM_DIM):
                vp = jnp.maximum(pc * ws[d] + bs[d], 0.0)
                slab[d, pl.ds(s, 16)] = jnp.where(pm, vp, 0.0)
                vc = jnp.maximum(cc * ws[d] + bs[d], 0.0)
                slab[NUM_DIM + d, pl.ds(s, 16)] = jnp.where(cm, vc, 0.0)
            ids = gv[pl.ds(s, 16)]
            cols = [plsc.load_gather(
                ctab, [jnp.full((16,), k, jnp.int32), ids])
                for k in range(CATE_DIM)]
            for k in range(CATE_DIM):
                slab[2 * NUM_DIM + k, pl.ds(s, 16)] = cols[k]
            return c2

        lax.fori_loop(0, 8, grp_a, 0)
        pltpu.sync_copy(slab, pa_h.at[l, :, pl.ds(b0, 128)])
        return carry

    lax.fori_loop(0, UNITS, unit_body, 0)


@functools.partial(
    pl.kernel,
    out_type=jax.ShapeDtypeStruct((L, OUT_DIM, B), jnp.float32),
    mesh=_mesh,
    compiler_params=_params,
    scratch_types=[
        pltpu.VMEM((128,), jnp.int32),
        pltpu.VMEM((128,), jnp.int32),
        pltpu.VMEM((128,), jnp.int32),
        pltpu.VMEM((128,), jnp.int32),
        pltpu.VMEM((128, 128), jnp.float32),
        pltpu.VMEM((128, 128), jnp.float32),
        pltpu.VMEM((OUT_DIM, 128), jnp.float32),
        pltpu.VMEM((OUT_DIM, 128), jnp.float32),
        pltpu.SemaphoreType.DMA,
        pltpu.SemaphoreType.DMA,
        pltpu.SemaphoreType.DMA,
        pltpu.SemaphoreType.DMA,
        pltpu.SemaphoreType.DMA,
        pltpu.SemaphoreType.DMA,
    ],
)
def _enc_item(itT_h, rt_h, pa_h, out_h,
              iv0, iv1, r0, r1, ir0, ir1, slabA, slabB,
              si0, si1, sg0, sg1, psA, psB):
    wid = lax.axis_index("s") * 2 + lax.axis_index("c")

    def lb(u):
        unit = u * NW + wid
        return unit // NBLK, (unit % NBLK) * 128

    def ids_src(u):
        l, b0 = lb(u)
        return itT_h.at[l, pl.ds(b0, 128)]

    def adjust(ivr, radj):
        def g8(g, c2):
            s = g * 16
            v = ivr[pl.ds(s, 16)]
            radj[pl.ds(s, 16)] = v - jnp.where(v >= SPLIT, SPLIT, 0)
            return c2

        lax.fori_loop(0, 8, g8, 0)

    def partial_descs(u, slab, ps):
        l, b0 = lb(u)
        return (
            (pa_h.at[l, pl.ds(0, NUM_DIM * 2), pl.ds(b0, 128)],
             slab.at[pl.ds(0, NUM_DIM * 2)], ps),
            (pa_h.at[l, pl.ds(NUM_DIM * 2, CATE_DIM), pl.ds(b0, 128)],
             slab.at[pl.ds(2 * NUM_DIM + ITEM_DIM, CATE_DIM)], ps),
        )

    def partial_start(u, slab, ps):
        for d in partial_descs(u, slab, ps):
            pltpu.async_copy(*d)

    def process(u, ivraw, r, ir, sg, slab, ps):
        l, b0 = lb(u)
        for d in partial_descs(u, slab, ps):
            pltpu.make_async_copy(*d).wait()
        pltpu.make_async_copy(rt_h.at[r], ir, sg).wait()

        def grp_b(g, c2):
            s = g * 16
            iv16 = lax.iota(jnp.int32, 16) + s
            base16 = jnp.where(ivraw[pl.ds(s, 16)] >= SPLIT, 64, 0)
            for cb in range(0, ITEM_DIM, 8):
                n = min(8, ITEM_DIM - cb)
                vs = [plsc.load_gather(ir, [iv16, base16 + (cb + j)])
                      for j in range(n)]
                for j in range(n):
                    slab[2 * NUM_DIM + cb + j, pl.ds(s, 16)] = vs[j]
            return c2

        lax.fori_loop(0, 8, grp_b, 0)
        pltpu.sync_copy(slab, out_h.at[l, :, pl.ds(b0, 128)])

    pltpu.sync_copy(ids_src(0), iv0)
    adjust(iv0, r0)
    pltpu.async_copy(rt_h.at[r0], ir0, sg0)
    partial_start(0, slabA, psA)
    pltpu.async_copy(ids_src(1), iv1, si1)

    def pair_body(t, carry):
        k0 = 2 * t
        k1 = k0 + 1
        pltpu.make_async_copy(ids_src(k1), iv1, si1).wait()
        adjust(iv1, r1)
        pltpu.async_copy(rt_h.at[r1], ir1, sg1)
        partial_start(k1, slabB, psB)
        process(k0, iv0, r0, ir0, sg0, slabA, psA)

        @pl.when(k0 + 2 < UNITS)
        def _():
            pltpu.async_copy(ids_src(k0 + 2), iv0, si0)

        @pl.when(k0 + 2 < UNITS)
        def _():
            pltpu.make_async_copy(ids_src(k0 + 2), iv0, si0).wait()
            adjust(iv0, r0)
            pltpu.async_copy(rt_h.at[r0], ir0, sg0)
            partial_start(k0 + 2, slabA, psA)

        process(k1, iv1, r1, ir1, sg1, slabB, psB)

        @pl.when(k1 + 2 < UNITS)
        def _():
            pltpu.async_copy(ids_src(k1 + 2), iv1, si1)

        return carry

    lax.fori_loop(0, UNITS // 2, pair_body, 0)


def kernel(price, ctr, item_id, cate_id, W_num, b_num, table_items,
           table_cates):
    wb = jnp.stack([W_num[0], b_num], axis=1).reshape(16)
    wb = jnp.broadcast_to(wb[:, None], (16, 16))
    partial = _enc_nc(price.T, ctr.T, cate_id.T, table_cates.T, wb)
    rtab = _relayout(table_items)
    outT = _enc_item(item_id.T, rtab, partial)
    return outT.transpose(2, 0, 1)
